# Initial kernel scaffold; baseline (speedup 1.0000x reference)
import jax
import jax.numpy as jnp
from jax.experimental import pallas as pl


def kernel(x, child_h, child_c, parent_idx, W, U_f, U_iuo):
    raise NotImplementedError("write your pallas kernel here")


# speedup vs baseline: 1.9974x; 1.9974x over previous
import functools

import jax
import jax.numpy as jnp
from jax import lax
from jax.experimental import pallas as pl
from jax.experimental.pallas import tpu as pltpu
from jax.experimental.pallas import tpu_sc as plsc

NC = 2
NS = 16
CK = 64


def _wx_body(x_ref, w_ref, f_ref, i_ref, u_ref, o_ref):
    d = f_ref.shape[1]
    wx = jnp.dot(x_ref[...], w_ref[...], preferred_element_type=jnp.float32)
    f_ref[...] = wx[:, :d]
    i_ref[...] = wx[:, d:2 * d]
    u_ref[...] = wx[:, 2 * d:3 * d]
    o_ref[...] = wx[:, 3 * d:]


def _wx_split(x, W):
    n, d = x.shape
    bn = 512
    grid = (n + bn - 1) // bn
    out = jax.ShapeDtypeStruct((n, d), jnp.float32)
    return pl.pallas_call(
        _wx_body,
        grid=(grid,),
        in_specs=[
            pl.BlockSpec((bn, d), lambda i: (i, 0)),
            pl.BlockSpec((d, 4 * d), lambda i: (0, 0)),
        ],
        out_specs=[pl.BlockSpec((bn, d), lambda i: (i, 0))] * 4,
        out_shape=[out] * 4,
    )(x, W)


def _ufh_body(h_ref, uf_ref, out_ref):
    out_ref[...] = jnp.dot(h_ref[...], uf_ref[...],
                           preferred_element_type=jnp.float32)


def _ufh(child_h, U_f):
    e, d = child_h.shape
    be = 2048
    grid = (e + be - 1) // be
    return pl.pallas_call(
        _ufh_body,
        grid=(grid,),
        in_specs=[
            pl.BlockSpec((be, d), lambda i: (i, 0)),
            pl.BlockSpec((d, d), lambda i: (0, 0)),
        ],
        out_specs=pl.BlockSpec((be, d), lambda i: (i, 0)),
        out_shape=jax.ShapeDtypeStruct((e, d), jnp.float32),
    )(child_h, U_f)


def _gates_body(ph_ref, pf_ref, wi_ref, wu_ref, wo_ref, uiuo_ref,
                h_ref, c_ref):
    d = wi_ref.shape[1]
    h_sum = ph_ref[0] + ph_ref[1]
    branch_f = pf_ref[0] + pf_ref[1]
    iuo = jnp.dot(h_sum, uiuo_ref[...], preferred_element_type=jnp.float32)
    gi = jax.nn.sigmoid(iuo[:, :d] + wi_ref[...])
    gu = jnp.tanh(iuo[:, d:2 * d] + wu_ref[...])
    go = jax.nn.sigmoid(iuo[:, 2 * d:] + wo_ref[...])
    new_c = gi * gu + branch_f
    c_ref[...] = new_c
    h_ref[...] = go * jnp.tanh(new_c)


def _gates(ph, pf, wix, wux, wox, U_iuo):
    n, d = wix.shape
    bn = 512
    grid = (n + bn - 1) // bn
    out = jax.ShapeDtypeStruct((n, d), jnp.float32)
    return pl.pallas_call(
        _gates_body,
        grid=(grid,),
        in_specs=[
            pl.BlockSpec((2, bn, d), lambda i: (0, i, 0)),
            pl.BlockSpec((2, bn, d), lambda i: (0, i, 0)),
            pl.BlockSpec((bn, d), lambda i: (i, 0)),
            pl.BlockSpec((bn, d), lambda i: (i, 0)),
            pl.BlockSpec((bn, d), lambda i: (i, 0)),
            pl.BlockSpec((d, 3 * d), lambda i: (0, 0)),
        ],
        out_specs=[pl.BlockSpec((bn, d), lambda i: (i, 0))] * 2,
        out_shape=[out] * 2,
    )(ph, pf, wix, wux, wox, U_iuo)


def _zero_vmem_rows(buf, nrows, d):
    zero = jnp.zeros((16,), jnp.float32)

    def body(r, _):
        for c8 in range(d // 16):
            buf[r, pl.ds(c8 * 16, 16)] = zero
        return 0

    lax.fori_loop(0, nrows, body, 0)


def _zero_stripe(acc, buf, sid, n_pad, d):
    stripe = n_pad // NS
    base = sid * stripe
    full, rem = divmod(stripe, CK)
    for k in range(full):
        pltpu.sync_copy(buf, acc.at[pl.ds(base + k * CK, CK)])
    if rem:
        pltpu.sync_copy(buf.at[pl.ds(0, rem)],
                        acc.at[pl.ds(base + full * CK, rem)])


def _write_stripe(acc, out, cid, sid, n_pad):
    stripe = n_pad // NS
    base = sid * stripe
    full, rem = divmod(stripe, CK)
    for k in range(full):
        pltpu.sync_copy(acc.at[pl.ds(base + k * CK, CK)],
                        out.at[cid, pl.ds(base + k * CK, CK)])
    if rem:
        pltpu.sync_copy(acc.at[pl.ds(base + full * CK, rem)],
                        out.at[cid, pl.ds(base + full * CK, rem)])


def _hsum_sc(n_pad, e, d, n_chunks, j_per_w):
    def body(ch_hbm, idx_hbm, out_hbm, idx_blk, rows, acc):
        cid = lax.axis_index("c")
        sid = lax.axis_index("s")
        w = cid * NS + sid

        _zero_vmem_rows(rows, CK, d)
        _zero_stripe(acc, rows, sid, n_pad, d)
        plsc.subcore_barrier()

        pltpu.sync_copy(idx_hbm.at[pl.ds(w * j_per_w, j_per_w)], idx_blk)

        def chunk(j, _):
            chunk_id = w * j_per_w + j

            @pl.when(chunk_id < n_chunks)
            def _():
                pltpu.sync_copy(ch_hbm.at[pl.ds(chunk_id * CK, CK)], rows)
                pltpu.sync_copy(rows, acc.at[idx_blk.at[j]], add=True)

            return 0

        lax.fori_loop(0, j_per_w, chunk, 0)
        plsc.subcore_barrier()
        _write_stripe(acc, out_hbm, cid, sid, n_pad)

    mesh = plsc.VectorSubcoreMesh(core_axis_name="c", subcore_axis_name="s")
    return pl.kernel(
        body,
        out_type=jax.ShapeDtypeStruct((NC, n_pad, d), jnp.float32),
        mesh=mesh,
        scratch_types=[
            pltpu.VMEM((j_per_w, CK), jnp.int32),
            pltpu.VMEM((CK, d), jnp.float32),
            pltpu.VMEM_SHARED((n_pad, d), jnp.float32),
        ],
    )


def _branchf_sc(n_pad, e, d, n_chunks, j_per_w):
    def body(wfx_hbm, ufh_hbm, cc_hbm, idx_hbm, out_hbm,
             idx_blk, g, u, c, acc, sem):
        cid = lax.axis_index("c")
        sid = lax.axis_index("s")
        w = cid * NS + sid

        _zero_vmem_rows(g, CK, d)
        _zero_stripe(acc, g, sid, n_pad, d)
        plsc.subcore_barrier()

        pltpu.sync_copy(idx_hbm.at[pl.ds(w * j_per_w, j_per_w)], idx_blk)

        def chunk(j, _):
            chunk_id = w * j_per_w + j

            @pl.when(chunk_id < n_chunks)
            def _():
                idxj = idx_blk.at[j]
                gather = pltpu.async_copy(wfx_hbm.at[idxj], g, sem)
                pltpu.sync_copy(ufh_hbm.at[pl.ds(chunk_id * CK, CK)], u)
                pltpu.sync_copy(cc_hbm.at[pl.ds(chunk_id * CK, CK)], c)
                gather.wait()

                def row(r, _):
                    for c8 in range(d // 16):
                        s = pl.ds(c8 * 16, 16)
                        xv = g[r, s] + u[r, s]
                        f = 1.0 / (1.0 + jnp.exp(-xv))
                        g[r, s] = f * c[r, s]
                    return 0

                lax.fori_loop(0, CK, row, 0)
                pltpu.sync_copy(g, acc.at[idxj], add=True)

            return 0

        lax.fori_loop(0, j_per_w, chunk, 0)
        plsc.subcore_barrier()
        _write_stripe(acc, out_hbm, cid, sid, n_pad)

    mesh = plsc.VectorSubcoreMesh(core_axis_name="c", subcore_axis_name="s")
    return pl.kernel(
        body,
        out_type=jax.ShapeDtypeStruct((NC, n_pad, d), jnp.float32),
        mesh=mesh,
        scratch_types=[
            pltpu.VMEM((j_per_w, CK), jnp.int32),
            pltpu.VMEM((CK, d), jnp.float32),
            pltpu.VMEM((CK, d), jnp.float32),
            pltpu.VMEM((CK, d), jnp.float32),
            pltpu.VMEM_SHARED((n_pad, d), jnp.float32),
            pltpu.SemaphoreType.DMA,
        ],
    )


@jax.jit
def kernel(x, child_h, child_c, parent_idx, W, U_f, U_iuo):
    n, d = x.shape
    e = child_h.shape[0]
    n_chunks = e // CK
    n_workers = NC * NS
    j_per_w = -(-n_chunks // n_workers)
    j_per_w = (j_per_w + 7) // 8 * 8
    n_pad = NS * ((-(-n // NS) + 7) // 8 * 8)

    pidx2d = parent_idx.reshape(n_chunks, CK)
    pad = n_workers * j_per_w - n_chunks
    if pad:
        pidx2d = jnp.pad(pidx2d, ((0, pad), (0, 0)))

    wfx, wix, wux, wox = _wx_split(x, W)
    ufh = _ufh(child_h, U_f)

    ph = _hsum_sc(n_pad, e, d, n_chunks, j_per_w)(child_h, pidx2d)
    pf = _branchf_sc(n_pad, e, d, n_chunks, j_per_w)(wfx, ufh, child_c, pidx2d)

    new_h, new_c = _gates(ph, pf, wix, wux, wox, U_iuo)
    return new_h, new_c
